# baseline (speedup 1.0000x reference)
import functools

import jax
import jax.numpy as jnp
from jax import lax
from jax.experimental import pallas as pl
from jax.experimental.pallas import tpu as pltpu
from jax.experimental.pallas import tpu_sc as plsc

E = 8
D = 1024
F = 4096
T = 2048

B = 64
NB = 40
NP = NB * B
FT = 1024
NF = F // FT

_NC, _NS = 2, 16
_NW = _NC * _NS


def _sc_row_gather(nrows_out, nrows_src):
    assert nrows_out % (8 * _NW) == 0
    b_per_w = nrows_out // _NW
    mesh = plsc.VectorSubcoreMesh(core_axis_name="c", subcore_axis_name="s")

    @functools.partial(
        pl.kernel, mesh=mesh,
        out_type=jax.ShapeDtypeStruct((nrows_out, D), jnp.float32),
        scratch_types=[
            pltpu.VMEM((b_per_w,), jnp.int32),
            pltpu.VMEM((b_per_w, D), jnp.float32),
            pltpu.SemaphoreType.DMA,
        ],
    )
    def gather_k(src_hbm, idx_hbm, out_hbm, idx_v, rows_v, sem):
        wid = lax.axis_index("s") * _NC + lax.axis_index("c")
        base = wid * b_per_w
        pltpu.sync_copy(idx_hbm.at[pl.ds(base, b_per_w)], idx_v)
        pltpu.async_copy(src_hbm.at[idx_v], rows_v, sem).wait()
        pltpu.sync_copy(rows_v, out_hbm.at[pl.ds(base, b_per_w)])

    return gather_k


def _sc_row_scatter():
    b_per_w = T // _NW
    mesh = plsc.VectorSubcoreMesh(core_axis_name="c", subcore_axis_name="s")

    @functools.partial(
        pl.kernel, mesh=mesh,
        out_type=jax.ShapeDtypeStruct((NP, D), jnp.float32),
        scratch_types=[
            pltpu.VMEM((b_per_w,), jnp.int32),
            pltpu.VMEM((b_per_w, D), jnp.float32),
            pltpu.SemaphoreType.DMA,
        ],
    )
    def scatter_k(src_hbm, idx_hbm, out_hbm, idx_v, rows_v, sem):
        wid = lax.axis_index("s") * _NC + lax.axis_index("c")
        base = wid * b_per_w
        pltpu.sync_copy(idx_hbm.at[pl.ds(base, b_per_w)], idx_v)
        pltpu.sync_copy(src_hbm.at[pl.ds(base, b_per_w)], rows_v)
        pltpu.async_copy(rows_v, out_hbm.at[idx_v], sem).wait()

    return scatter_k


_scatter_in = _sc_row_scatter()
_gather_out = _sc_row_gather(T, NP)


def _ffn_body(be_ref, xs_ref, w1_ref, b1_ref, w2_ref, b2_ref, out_ref):
    e = pl.program_id(0)
    f = pl.program_id(1)
    w1 = w1_ref[0]
    w2 = w2_ref[0]
    b1v = b1_ref[0]
    b2v = b2_ref[0]
    for b in range(NB):
        @pl.when(be_ref[b] == e)
        def _(b=b):
            xb = xs_ref[pl.ds(b * B, B), :]
            h = jnp.maximum(
                jnp.dot(xb, w1, preferred_element_type=jnp.float32) + b1v, 0.0)
            contrib = jnp.dot(h, w2, preferred_element_type=jnp.float32)

            @pl.when(f == 0)
            def _():
                out_ref[pl.ds(b * B, B), :] = contrib

            @pl.when(f > 0)
            def _():
                out_ref[pl.ds(b * B, B), :] += contrib

            @pl.when(f == NF - 1)
            def _():
                out_ref[pl.ds(b * B, B), :] += b2v


_ffn = pl.pallas_call(
    _ffn_body,
    grid_spec=pltpu.PrefetchScalarGridSpec(
        num_scalar_prefetch=1,
        grid=(E, NF),
        in_specs=[
            pl.BlockSpec((NP, D), lambda e, f, be: (0, 0)),
            pl.BlockSpec((1, D, FT), lambda e, f, be: (e, 0, f)),
            pl.BlockSpec((1, 1, FT), lambda e, f, be: (e, 0, f)),
            pl.BlockSpec((1, FT, D), lambda e, f, be: (e, f, 0)),
            pl.BlockSpec((1, 1, D), lambda e, f, be: (e, 0, 0)),
        ],
        out_specs=pl.BlockSpec((NP, D), lambda e, f, be: (0, 0)),
    ),
    out_shape=jax.ShapeDtypeStruct((NP, D), jnp.float32),
)


def kernel(input, W1, b1, W2, b2, Wg, bg):
    logits = input @ Wg + bg
    p = jax.nn.softmax(logits, axis=-1)
    skey = jax.random.fold_in(jax.random.key(42), 7)
    sample = jax.random.categorical(skey, jnp.log(p + 1e-20), axis=-1)
    sample = jax.lax.stop_gradient(sample)

    onehot = (sample[:, None] == jnp.arange(E, dtype=sample.dtype)[None, :]
              ).astype(jnp.int32)
    inc = jnp.cumsum(onehot, axis=0)
    counts = inc[-1, :]
    nblk = (counts + B - 1) // B
    blk_start = jnp.concatenate(
        [jnp.zeros(1, jnp.int32), jnp.cumsum(nblk)[:-1].astype(jnp.int32)])
    pstart = blk_start * B
    rank = jnp.sum(inc * onehot, axis=1) - 1
    pos = jnp.sum(pstart[None, :] * onehot, axis=1) + rank
    block_expert = (jnp.sum(
        jnp.arange(NB, dtype=jnp.int32)[:, None] >= blk_start[None, :],
        axis=1) - 1).astype(jnp.int32)

    xs = _scatter_in(input, pos)
    ys = _ffn(block_expert, xs, W1, b1.reshape(E, 1, F),
              W2, b2.reshape(E, 1, D))
    return _gather_out(ys, pos)


# speedup vs baseline: 1.2688x; 1.2688x over previous
import functools

import jax
import jax.numpy as jnp
from jax import lax
from jax.experimental import pallas as pl
from jax.experimental.pallas import tpu as pltpu
from jax.experimental.pallas import tpu_sc as plsc

E = 8
D = 1024
F = 4096
T = 2048

B = 256
NB = 16
NP = NB * B
FT = 1024
NF = F // FT

_NC, _NS = 2, 16
_NW = _NC * _NS


def _sc_row_gather(nrows_out, nrows_src):
    assert nrows_out % (8 * _NW) == 0
    b_per_w = nrows_out // _NW
    mesh = plsc.VectorSubcoreMesh(core_axis_name="c", subcore_axis_name="s")

    @functools.partial(
        pl.kernel, mesh=mesh,
        out_type=jax.ShapeDtypeStruct((nrows_out, D), jnp.float32),
        scratch_types=[
            pltpu.VMEM((b_per_w,), jnp.int32),
            pltpu.VMEM((b_per_w, D), jnp.float32),
            pltpu.SemaphoreType.DMA,
        ],
    )
    def gather_k(src_hbm, idx_hbm, out_hbm, idx_v, rows_v, sem):
        wid = lax.axis_index("s") * _NC + lax.axis_index("c")
        base = wid * b_per_w
        pltpu.sync_copy(idx_hbm.at[pl.ds(base, b_per_w)], idx_v)
        pltpu.async_copy(src_hbm.at[idx_v], rows_v, sem).wait()
        pltpu.sync_copy(rows_v, out_hbm.at[pl.ds(base, b_per_w)])

    return gather_k


def _sc_row_scatter():
    b_per_w = T // _NW
    mesh = plsc.VectorSubcoreMesh(core_axis_name="c", subcore_axis_name="s")

    @functools.partial(
        pl.kernel, mesh=mesh,
        out_type=jax.ShapeDtypeStruct((NP, D), jnp.float32),
        scratch_types=[
            pltpu.VMEM((b_per_w,), jnp.int32),
            pltpu.VMEM((b_per_w, D), jnp.float32),
            pltpu.SemaphoreType.DMA,
        ],
    )
    def scatter_k(src_hbm, idx_hbm, out_hbm, idx_v, rows_v, sem):
        wid = lax.axis_index("s") * _NC + lax.axis_index("c")
        base = wid * b_per_w
        pltpu.sync_copy(idx_hbm.at[pl.ds(base, b_per_w)], idx_v)
        pltpu.sync_copy(src_hbm.at[pl.ds(base, b_per_w)], rows_v)
        pltpu.async_copy(rows_v, out_hbm.at[idx_v], sem).wait()

    return scatter_k


_scatter_in = _sc_row_scatter()
_gather_out = _sc_row_gather(T, NP)


def _ffn_body(be_ref, xs_ref, w1_ref, b1_ref, w2_ref, b2_ref, out_ref):
    e = pl.program_id(0)
    f = pl.program_id(1)
    w1 = w1_ref[0]
    w2 = w2_ref[0]
    b1v = b1_ref[0]
    b2v = b2_ref[0]
    for b in range(NB):
        @pl.when(be_ref[b] == e)
        def _(b=b):
            xb = xs_ref[pl.ds(b * B, B), :]
            h = jnp.maximum(
                jnp.dot(xb, w1, preferred_element_type=jnp.float32) + b1v, 0.0)
            contrib = jnp.dot(h, w2, preferred_element_type=jnp.float32)

            @pl.when(f == 0)
            def _():
                out_ref[pl.ds(b * B, B), :] = contrib

            @pl.when(f > 0)
            def _():
                out_ref[pl.ds(b * B, B), :] += contrib

            @pl.when(f == NF - 1)
            def _():
                out_ref[pl.ds(b * B, B), :] += b2v


_ffn = pl.pallas_call(
    _ffn_body,
    grid_spec=pltpu.PrefetchScalarGridSpec(
        num_scalar_prefetch=1,
        grid=(E, NF),
        in_specs=[
            pl.BlockSpec((NP, D), lambda e, f, be: (0, 0)),
            pl.BlockSpec((1, D, FT), lambda e, f, be: (e, 0, f)),
            pl.BlockSpec((1, 1, FT), lambda e, f, be: (e, 0, f)),
            pl.BlockSpec((1, FT, D), lambda e, f, be: (e, f, 0)),
            pl.BlockSpec((1, 1, D), lambda e, f, be: (e, 0, 0)),
        ],
        out_specs=pl.BlockSpec((NP, D), lambda e, f, be: (0, 0)),
    ),
    out_shape=jax.ShapeDtypeStruct((NP, D), jnp.float32),
)


def kernel(input, W1, b1, W2, b2, Wg, bg):
    logits = input @ Wg + bg
    p = jax.nn.softmax(logits, axis=-1)
    skey = jax.random.fold_in(jax.random.key(42), 7)
    sample = jax.random.categorical(skey, jnp.log(p + 1e-20), axis=-1)
    sample = jax.lax.stop_gradient(sample)

    onehot = (sample[:, None] == jnp.arange(E, dtype=sample.dtype)[None, :]
              ).astype(jnp.int32)
    inc = jnp.cumsum(onehot, axis=0)
    counts = inc[-1, :]
    nblk = (counts + B - 1) // B
    blk_start = jnp.concatenate(
        [jnp.zeros(1, jnp.int32), jnp.cumsum(nblk)[:-1].astype(jnp.int32)])
    pstart = blk_start * B
    rank = jnp.sum(inc * onehot, axis=1) - 1
    pos = jnp.sum(pstart[None, :] * onehot, axis=1) + rank
    block_expert = (jnp.sum(
        jnp.arange(NB, dtype=jnp.int32)[:, None] >= blk_start[None, :],
        axis=1) - 1).astype(jnp.int32)

    xs = _scatter_in(input, pos)
    ys = _ffn(block_expert, xs, W1, b1.reshape(E, 1, F),
              W2, b2.reshape(E, 1, D))
    return _gather_out(ys, pos)


# speedup vs baseline: 1.3293x; 1.0476x over previous
import functools

import jax
import jax.numpy as jnp
from jax import lax
from jax.experimental import pallas as pl
from jax.experimental.pallas import tpu as pltpu
from jax.experimental.pallas import tpu_sc as plsc

E = 8
D = 1024
F = 4096
T = 2048

B = 128
NB = 24
NP = NB * B
FT = 1024
NF = F // FT

_NC, _NS = 2, 16
_NW = _NC * _NS


def _sc_row_gather(nrows_out, nrows_src):
    assert nrows_out % (8 * _NW) == 0
    b_per_w = nrows_out // _NW
    mesh = plsc.VectorSubcoreMesh(core_axis_name="c", subcore_axis_name="s")

    @functools.partial(
        pl.kernel, mesh=mesh,
        out_type=jax.ShapeDtypeStruct((nrows_out, D), jnp.float32),
        scratch_types=[
            pltpu.VMEM((b_per_w,), jnp.int32),
            pltpu.VMEM((b_per_w, D), jnp.float32),
            pltpu.SemaphoreType.DMA,
        ],
    )
    def gather_k(src_hbm, idx_hbm, out_hbm, idx_v, rows_v, sem):
        wid = lax.axis_index("s") * _NC + lax.axis_index("c")
        base = wid * b_per_w
        pltpu.sync_copy(idx_hbm.at[pl.ds(base, b_per_w)], idx_v)
        pltpu.async_copy(src_hbm.at[idx_v], rows_v, sem).wait()
        pltpu.sync_copy(rows_v, out_hbm.at[pl.ds(base, b_per_w)])

    return gather_k


def _sc_row_scatter():
    b_per_w = T // _NW
    mesh = plsc.VectorSubcoreMesh(core_axis_name="c", subcore_axis_name="s")

    @functools.partial(
        pl.kernel, mesh=mesh,
        out_type=jax.ShapeDtypeStruct((NP, D), jnp.float32),
        scratch_types=[
            pltpu.VMEM((b_per_w,), jnp.int32),
            pltpu.VMEM((b_per_w, D), jnp.float32),
            pltpu.SemaphoreType.DMA,
        ],
    )
    def scatter_k(src_hbm, idx_hbm, out_hbm, idx_v, rows_v, sem):
        wid = lax.axis_index("s") * _NC + lax.axis_index("c")
        base = wid * b_per_w
        pltpu.sync_copy(idx_hbm.at[pl.ds(base, b_per_w)], idx_v)
        pltpu.sync_copy(src_hbm.at[pl.ds(base, b_per_w)], rows_v)
        pltpu.async_copy(rows_v, out_hbm.at[idx_v], sem).wait()

    return scatter_k


_scatter_in = _sc_row_scatter()
_gather_out = _sc_row_gather(T, NP)


def _ffn_body(be_ref, xs_ref, w1_ref, b1_ref, w2_ref, b2_ref, out_ref):
    e = pl.program_id(0)
    f = pl.program_id(1)
    w1 = w1_ref[0]
    w2 = w2_ref[0]
    b1v = b1_ref[0]
    b2v = b2_ref[0]
    for b in range(NB):
        @pl.when(be_ref[b] == e)
        def _(b=b):
            xb = xs_ref[pl.ds(b * B, B), :]
            h = jnp.maximum(
                jnp.dot(xb, w1, preferred_element_type=jnp.float32) + b1v, 0.0)
            contrib = jnp.dot(h, w2, preferred_element_type=jnp.float32)

            @pl.when(f == 0)
            def _():
                out_ref[pl.ds(b * B, B), :] = contrib

            @pl.when(f > 0)
            def _():
                out_ref[pl.ds(b * B, B), :] += contrib

            @pl.when(f == NF - 1)
            def _():
                out_ref[pl.ds(b * B, B), :] += b2v


_ffn = pl.pallas_call(
    _ffn_body,
    grid_spec=pltpu.PrefetchScalarGridSpec(
        num_scalar_prefetch=1,
        grid=(E, NF),
        in_specs=[
            pl.BlockSpec((NP, D), lambda e, f, be: (0, 0)),
            pl.BlockSpec((1, D, FT), lambda e, f, be: (e, 0, f)),
            pl.BlockSpec((1, 1, FT), lambda e, f, be: (e, 0, f)),
            pl.BlockSpec((1, FT, D), lambda e, f, be: (e, f, 0)),
            pl.BlockSpec((1, 1, D), lambda e, f, be: (e, 0, 0)),
        ],
        out_specs=pl.BlockSpec((NP, D), lambda e, f, be: (0, 0)),
    ),
    out_shape=jax.ShapeDtypeStruct((NP, D), jnp.float32),
)


def kernel(input, W1, b1, W2, b2, Wg, bg):
    logits = input @ Wg + bg
    p = jax.nn.softmax(logits, axis=-1)
    skey = jax.random.fold_in(jax.random.key(42), 7)
    sample = jax.random.categorical(skey, jnp.log(p + 1e-20), axis=-1)
    sample = jax.lax.stop_gradient(sample)

    onehot = (sample[:, None] == jnp.arange(E, dtype=sample.dtype)[None, :]
              ).astype(jnp.int32)
    inc = jnp.cumsum(onehot, axis=0)
    counts = inc[-1, :]
    nblk = (counts + B - 1) // B
    blk_start = jnp.concatenate(
        [jnp.zeros(1, jnp.int32), jnp.cumsum(nblk)[:-1].astype(jnp.int32)])
    pstart = blk_start * B
    rank = jnp.sum(inc * onehot, axis=1) - 1
    pos = jnp.sum(pstart[None, :] * onehot, axis=1) + rank
    block_expert = (jnp.sum(
        jnp.arange(NB, dtype=jnp.int32)[:, None] >= blk_start[None, :],
        axis=1) - 1).astype(jnp.int32)

    xs = _scatter_in(input, pos)
    ys = _ffn(block_expert, xs, W1, b1.reshape(E, 1, F),
              W2, b2.reshape(E, 1, D))
    return _gather_out(ys, pos)
